# baseline (speedup 1.0000x reference)
import jax
import jax.numpy as jnp
from jax.experimental import pallas as pl
from jax.experimental.pallas import tpu as pltpu

N = 1024
IN_DIM = 128
HID = 256
LAYERS = 2
ACH = 128
NCH = N // ACH


def _glstm_kernel(x_hbm, w_in_hbm, b_in_hbm, adj_hbm, wx_hbm, wh_hbm, wm_hbm,
                  bc_hbm, wo_hbm, bo_hbm, out_ref,
                  x_v, w_in_v, b_in_v, adj_v, wx_v, wh_v, wm_v, bc_v, wo_v,
                  bo_v, xh_v, agg_v, sems):
    cp_x = pltpu.make_async_copy(x_hbm, x_v, sems.at[0])
    cp_win = pltpu.make_async_copy(w_in_hbm, w_in_v, sems.at[1])
    cp_bin = pltpu.make_async_copy(b_in_hbm, b_in_v, sems.at[2])
    cp_adj = [
        pltpu.make_async_copy(adj_hbm.at[pl.ds(k * ACH, ACH), :],
                              adj_v.at[pl.ds(k * ACH, ACH), :], sems.at[3 + k])
        for k in range(NCH)
    ]
    cp_wx = pltpu.make_async_copy(wx_hbm, wx_v, sems.at[3 + NCH])
    cp_wm = pltpu.make_async_copy(wm_hbm, wm_v, sems.at[4 + NCH])
    cp_wh = pltpu.make_async_copy(wh_hbm, wh_v, sems.at[5 + NCH])
    cp_bc = pltpu.make_async_copy(bc_hbm, bc_v, sems.at[6 + NCH])
    cp_wo = pltpu.make_async_copy(wo_hbm, wo_v, sems.at[7 + NCH])
    cp_bo = pltpu.make_async_copy(bo_hbm, bo_v, sems.at[8 + NCH])

    cp_x.start()
    cp_win.start()
    cp_bin.start()
    for cp in cp_adj:
        cp.start()
    cp_wx.start()
    cp_wm.start()
    cp_wh.start()
    cp_bc.start()
    cp_wo.start()
    cp_bo.start()

    cp_x.wait()
    cp_win.wait()
    cp_bin.wait()
    xh_v[...] = (
        jnp.dot(x_v[...], w_in_v[...], preferred_element_type=jnp.float32)
        + b_in_v[...]
    )
    xh_bf = xh_v[...].astype(jnp.bfloat16)

    for k in range(NCH):
        cp_adj[k].wait()
        blk = adj_v[pl.ds(k * ACH, ACH), :]
        mask = blk != 0
        a = mask.astype(jnp.bfloat16)
        deg = jnp.sum(mask.astype(jnp.float32), axis=1, keepdims=True)
        s = jnp.dot(a, xh_bf, preferred_element_type=jnp.float32)
        agg_v[pl.ds(k * ACH, ACH), :] = s / jnp.maximum(deg, 1.0)

    cp_wx.wait()
    cp_wm.wait()
    cp_bc.wait()
    xh = xh_v[...]
    base = (
        jnp.dot(xh, wx_v[...], preferred_element_type=jnp.float32)
        + jnp.dot(agg_v[...], wm_v[...], preferred_element_type=jnp.float32)
        + bc_v[...]
    )

    cp_wh.wait()
    h = xh
    c = xh
    for _ in range(LAYERS):
        gates = base + jnp.dot(h, wh_v[...], preferred_element_type=jnp.float32)
        i_g = gates[:, 0 * HID:1 * HID]
        f_g = gates[:, 1 * HID:2 * HID]
        o_g = gates[:, 2 * HID:3 * HID]
        g_g = gates[:, 3 * HID:4 * HID]
        sig_f = 0.5 * jnp.tanh(f_g * 0.5) + 0.5
        sig_i = 0.5 * jnp.tanh(i_g * 0.5) + 0.5
        sig_o = 0.5 * jnp.tanh(o_g * 0.5) + 0.5
        c = sig_f * c + sig_i * jnp.tanh(g_g)
        h = sig_o * jnp.tanh(c)

    cp_wo.wait()
    cp_bo.wait()
    out_ref[...] = (
        jnp.dot(h, wo_v[...], preferred_element_type=jnp.float32) + bo_v[...]
    )


def kernel(x, edge_indexs, edgenum, W_in, b_in, W_x, W_h, W_m, b_cell, W_out, b_out):
    x2 = x.reshape(N, IN_DIM)
    adj = edge_indexs.reshape(N, N)

    out = pl.pallas_call(
        _glstm_kernel,
        in_specs=[pl.BlockSpec(memory_space=pl.ANY)] * 10,
        out_shape=jax.ShapeDtypeStruct((N, 1), jnp.float32),
        scratch_shapes=[
            pltpu.VMEM((N, IN_DIM), jnp.float32),
            pltpu.VMEM((IN_DIM, HID), jnp.float32),
            pltpu.VMEM((1, HID), jnp.float32),
            pltpu.VMEM((N, N), jnp.int32),
            pltpu.VMEM((HID, 4 * HID), jnp.float32),
            pltpu.VMEM((HID, 4 * HID), jnp.float32),
            pltpu.VMEM((HID, 4 * HID), jnp.float32),
            pltpu.VMEM((1, 4 * HID), jnp.float32),
            pltpu.VMEM((HID, 1), jnp.float32),
            pltpu.VMEM((1, 1), jnp.float32),
            pltpu.VMEM((N, HID), jnp.float32),
            pltpu.VMEM((N, HID), jnp.float32),
            pltpu.SemaphoreType.DMA((9 + NCH,)),
        ],
    )(x2, W_in, b_in.reshape(1, HID), adj, W_x, W_h, W_m,
      b_cell.reshape(1, 4 * HID), W_out, b_out.reshape(1, 1))

    return out.reshape(1, N, 1)


# speedup vs baseline: 1.0755x; 1.0755x over previous
import jax
import jax.numpy as jnp
from jax.experimental import pallas as pl
from jax.experimental.pallas import tpu as pltpu

N = 1024
IN_DIM = 128
HID = 256
LAYERS = 2
ACH = 512
NCH = N // ACH


def _glstm_kernel(x_hbm, w_in_hbm, b_in_hbm, adj_hbm, wx_hbm, wh_hbm, wm_hbm,
                  bc_hbm, wo_hbm, bo_hbm, out_ref,
                  x_v, w_in_v, b_in_v, adj_v, wx_v, wh_v, wm_v, bc_v, wo_v,
                  bo_v, xh_v, agg_v, sems):
    cp_x = pltpu.make_async_copy(x_hbm, x_v, sems.at[0])
    cp_win = pltpu.make_async_copy(w_in_hbm, w_in_v, sems.at[1])
    cp_bin = pltpu.make_async_copy(b_in_hbm, b_in_v, sems.at[2])
    cp_adj = [
        pltpu.make_async_copy(adj_hbm.at[pl.ds(k * ACH, ACH), :],
                              adj_v.at[pl.ds(k * ACH, ACH), :], sems.at[3 + k])
        for k in range(NCH)
    ]
    cp_wx = pltpu.make_async_copy(wx_hbm, wx_v, sems.at[3 + NCH])
    cp_wm = pltpu.make_async_copy(wm_hbm, wm_v, sems.at[4 + NCH])
    cp_wh = pltpu.make_async_copy(wh_hbm, wh_v, sems.at[5 + NCH])
    cp_bc = pltpu.make_async_copy(bc_hbm, bc_v, sems.at[6 + NCH])
    cp_wo = pltpu.make_async_copy(wo_hbm, wo_v, sems.at[7 + NCH])
    cp_bo = pltpu.make_async_copy(bo_hbm, bo_v, sems.at[8 + NCH])

    cp_x.start()
    cp_win.start()
    cp_bin.start()
    for cp in cp_adj:
        cp.start()
    cp_wx.start()
    cp_wm.start()
    cp_wh.start()
    cp_bc.start()
    cp_wo.start()
    cp_bo.start()

    cp_x.wait()
    cp_win.wait()
    cp_bin.wait()
    xh_v[...] = (
        jnp.dot(x_v[...], w_in_v[...], preferred_element_type=jnp.float32)
        + b_in_v[...]
    )
    xh_bf = xh_v[...].astype(jnp.bfloat16)

    for k in range(NCH):
        cp_adj[k].wait()
        blk = adj_v[pl.ds(k * ACH, ACH), :]
        mask = blk != 0
        a = mask.astype(jnp.bfloat16)
        deg = jnp.sum(mask.astype(jnp.float32), axis=1, keepdims=True)
        s = jnp.dot(a, xh_bf, preferred_element_type=jnp.float32)
        agg_v[pl.ds(k * ACH, ACH), :] = s / jnp.maximum(deg, 1.0)

    cp_wx.wait()
    cp_wm.wait()
    cp_bc.wait()
    xh = xh_v[...]
    base = (
        jnp.dot(xh, wx_v[...], preferred_element_type=jnp.float32)
        + jnp.dot(agg_v[...], wm_v[...], preferred_element_type=jnp.float32)
        + bc_v[...]
    )

    cp_wh.wait()
    h = xh
    c = xh
    for _ in range(LAYERS):
        gates = base + jnp.dot(h, wh_v[...], preferred_element_type=jnp.float32)
        i_g = gates[:, 0 * HID:1 * HID]
        f_g = gates[:, 1 * HID:2 * HID]
        o_g = gates[:, 2 * HID:3 * HID]
        g_g = gates[:, 3 * HID:4 * HID]
        sig_f = 0.5 * jnp.tanh(f_g * 0.5) + 0.5
        sig_i = 0.5 * jnp.tanh(i_g * 0.5) + 0.5
        sig_o = 0.5 * jnp.tanh(o_g * 0.5) + 0.5
        c = sig_f * c + sig_i * jnp.tanh(g_g)
        h = sig_o * jnp.tanh(c)

    cp_wo.wait()
    cp_bo.wait()
    out_ref[...] = (
        jnp.dot(h, wo_v[...], preferred_element_type=jnp.float32) + bo_v[...]
    )


def kernel(x, edge_indexs, edgenum, W_in, b_in, W_x, W_h, W_m, b_cell, W_out, b_out):
    x2 = x.reshape(N, IN_DIM)
    adj = edge_indexs.reshape(N, N)

    out = pl.pallas_call(
        _glstm_kernel,
        in_specs=[pl.BlockSpec(memory_space=pl.ANY)] * 10,
        out_shape=jax.ShapeDtypeStruct((N, 1), jnp.float32),
        scratch_shapes=[
            pltpu.VMEM((N, IN_DIM), jnp.float32),
            pltpu.VMEM((IN_DIM, HID), jnp.float32),
            pltpu.VMEM((1, HID), jnp.float32),
            pltpu.VMEM((N, N), jnp.int32),
            pltpu.VMEM((HID, 4 * HID), jnp.float32),
            pltpu.VMEM((HID, 4 * HID), jnp.float32),
            pltpu.VMEM((HID, 4 * HID), jnp.float32),
            pltpu.VMEM((1, 4 * HID), jnp.float32),
            pltpu.VMEM((HID, 1), jnp.float32),
            pltpu.VMEM((1, 1), jnp.float32),
            pltpu.VMEM((N, HID), jnp.float32),
            pltpu.VMEM((N, HID), jnp.float32),
            pltpu.SemaphoreType.DMA((9 + NCH,)),
        ],
    )(x2, W_in, b_in.reshape(1, HID), adj, W_x, W_h, W_m,
      b_cell.reshape(1, 4 * HID), W_out, b_out.reshape(1, 1))

    return out.reshape(1, N, 1)
